# baseline (speedup 1.0000x reference)
import functools

import jax
import jax.numpy as jnp
from jax import lax
from jax.experimental import pallas as pl
from jax.experimental.pallas import tpu as pltpu
from jax.experimental.pallas import tpu_sc as plsc

_L = 16

_mesh = plsc.VectorSubcoreMesh(
    core_axis_name="c", subcore_axis_name="s", num_cores=1)


@functools.partial(
    pl.kernel,
    mesh=_mesh,
    out_type=jax.ShapeDtypeStruct((_L,), jnp.float32),
    scratch_types=[
        pltpu.VMEM((_L,), jnp.float32),
    ],
)
def _probe(flat_hbm, label_hbm, out_hbm, vec_v):
    sid = lax.axis_index("s")

    @pl.when(sid == 0)
    def _():
        vec_v[...] = jnp.zeros((_L,), jnp.float32)
        pltpu.sync_copy(vec_v, out_hbm)


def kernel(feature, label):
    flat = feature.T.reshape(-1)
    out = _probe(flat, label)
    return out[0]


# speedup vs baseline: 3.2118x; 3.2118x over previous
import functools

import jax
import jax.numpy as jnp
from jax import lax
from jax.experimental import pallas as pl
from jax.experimental.pallas import tpu as pltpu
from jax.experimental.pallas import tpu_sc as plsc

_SCALE = 64.0
_B = 16384
_C = 1000
_L = 16
_NS = 16
_BPW = _B // _NS
_ICH = 128
_NCH = _BPW // _ICH
_COEF = 2.0 / (_B * _SCALE)

_mesh = plsc.VectorSubcoreMesh(
    core_axis_name="c", subcore_axis_name="s", num_cores=1)


@functools.partial(
    pl.kernel,
    mesh=_mesh,
    out_type=jax.ShapeDtypeStruct((_L,), jnp.float32),
    scratch_types=[
        pltpu.VMEM((_BPW,), jnp.int32),
        pltpu.VMEM((_NCH, _ICH), jnp.int32),
        pltpu.VMEM((_NCH, _ICH), jnp.float32),
        pltpu.VMEM((_L,), jnp.float32),
        pltpu.VMEM((_NS * _L,), jnp.float32),
        pltpu.VMEM_SHARED((_NS * _L,), jnp.float32),
        pltpu.SemaphoreType.DMA,
    ],
)
def _center_kernel(flat_hbm, label_hbm, out_hbm, lab_v, idx_v, val_v,
                   vec_v, all_v, shared, sem):
    sid = lax.axis_index("s")
    base = sid * _BPW

    pltpu.sync_copy(label_hbm.at[pl.ds(base, _BPW)], lab_v)

    lanes = lax.iota(jnp.int32, _L)
    row_hi = (base >> 7) << 10
    copies = []
    for k in range(_NCH):
        for j in range(_ICH // _L):
            off = k * _ICH + j * _L
            lab = lab_v[pl.ds(off, _L)]
            row_part = row_hi + ((off >> 7) << 10) + (off & 127)
            idx_v[k, pl.ds(j * _L, _L)] = (
                ((lab >> 3) << 17) + ((lab & 7) << 7) + (row_part + lanes))
        copies.append(
            pltpu.async_copy(flat_hbm.at[idx_v.at[k]], val_v.at[k], sem))

    acc = jnp.zeros((_L,), jnp.float32)
    for k in range(_NCH):
        copies[k].wait()
        for j in range(_ICH // _L):
            acc = acc + val_v[k, pl.ds(j * _L, _L)]

    vec_v[...] = acc
    pltpu.sync_copy(vec_v, shared.at[pl.ds(sid * _L, _L)])
    plsc.subcore_barrier()

    @pl.when(sid == 0)
    def _():
        pltpu.sync_copy(shared, all_v)
        tot = jnp.zeros((_L,), jnp.float32)
        for t in range(_NS):
            tot = tot + all_v[pl.ds(t * _L, _L)]
        s = tot[0]
        for i in range(1, _L):
            s = s + tot[i]
        loss = 2.0 - _COEF * s
        vec_v[...] = jnp.broadcast_to(loss, (_L,))
        pltpu.sync_copy(vec_v, out_hbm)


def kernel(feature, label):
    b, c = feature.shape
    flat = feature.T.reshape(c // 8, 8, b // 128, 128).transpose(
        0, 2, 1, 3).reshape(-1)
    out = _center_kernel(flat, label)
    return out[0]


# speedup vs baseline: 3.2660x; 1.0169x over previous
import functools

import jax
import jax.numpy as jnp
from jax import lax
from jax.experimental import pallas as pl
from jax.experimental.pallas import tpu as pltpu
from jax.experimental.pallas import tpu_sc as plsc

_SCALE = 64.0
_B = 16384
_C = 1000
_L = 16
_NS = 16
_BPW = _B // _NS
_ICH = 128
_NCH = _BPW // _ICH
_COEF = 2.0 / (_B * _SCALE)

_mesh = plsc.VectorSubcoreMesh(
    core_axis_name="c", subcore_axis_name="s", num_cores=1)


@functools.partial(
    pl.kernel,
    mesh=_mesh,
    out_type=jax.ShapeDtypeStruct((_L,), jnp.float32),
    scratch_types=[
        pltpu.VMEM((_BPW,), jnp.int32),
        pltpu.VMEM((_NCH, _ICH), jnp.int32),
        pltpu.VMEM((_NCH, _ICH), jnp.float32),
        pltpu.VMEM((_L,), jnp.float32),
        pltpu.VMEM((_NS * _L,), jnp.float32),
        pltpu.VMEM_SHARED((_NS * _L,), jnp.float32),
        pltpu.SemaphoreType.DMA,
    ],
)
def _center_kernel(flat_hbm, label_hbm, out_hbm, lab_v, idx_v, val_v,
                   vec_v, all_v, shared, sem):
    sid = lax.axis_index("s")
    base = sid * _BPW

    pltpu.sync_copy(label_hbm.at[pl.ds(base, _BPW)], lab_v)

    lanes = lax.iota(jnp.int32, _L)
    row_hi = (base >> 7) << 10

    def _fire(k, _):
        for j in range(_ICH // _L):
            off = k * _ICH + j * _L
            lab = lab_v[pl.ds(off, _L)]
            row_part = row_hi + ((k >> 0) << 10) + j * _L + lanes
            idx_v[k, pl.ds(j * _L, _L)] = (
                ((lab >> 3) << 17) + ((lab & 7) << 7) + row_part)
        pltpu.async_copy(flat_hbm.at[idx_v.at[k]], val_v.at[k], sem)
        return 0

    lax.fori_loop(0, _NCH, _fire, 0, unroll=False)

    def _drain(k, acc):
        pltpu.make_async_copy(
            flat_hbm.at[idx_v.at[k]], val_v.at[k], sem).wait()
        for j in range(_ICH // _L):
            acc = acc + val_v[k, pl.ds(j * _L, _L)]
        return acc

    acc = lax.fori_loop(
        0, _NCH, _drain, jnp.zeros((_L,), jnp.float32), unroll=False)

    vec_v[...] = acc
    pltpu.sync_copy(vec_v, shared.at[pl.ds(sid * _L, _L)])
    plsc.subcore_barrier()

    @pl.when(sid == 0)
    def _():
        pltpu.sync_copy(shared, all_v)
        tot = jnp.zeros((_L,), jnp.float32)
        for t in range(_NS):
            tot = tot + all_v[pl.ds(t * _L, _L)]
        s = tot[0]
        for i in range(1, _L):
            s = s + tot[i]
        loss = 2.0 - _COEF * s
        vec_v[...] = jnp.broadcast_to(loss, (_L,))
        pltpu.sync_copy(vec_v, out_hbm)


def kernel(feature, label):
    b, c = feature.shape
    flat = feature.T.reshape(c // 8, 8, b // 128, 128).transpose(
        0, 2, 1, 3).reshape(-1)
    out = _center_kernel(flat, label)
    return out[0]
